# Initial kernel scaffold; baseline (speedup 1.0000x reference)
import jax
import jax.numpy as jnp
from jax.experimental import pallas as pl


def kernel(observations, edge_index, W_fc, b_fc, W_attn, b_attn):
    raise NotImplementedError("write your pallas kernel here")


# speedup vs baseline: 1.4052x; 1.4052x over previous
import functools

import jax
import jax.numpy as jnp
from jax import lax
from jax.experimental import pallas as pl
from jax.experimental.pallas import tpu as pltpu
from jax.experimental.pallas import tpu_sc as plsc

N = 10000
E = 160000
D = 256

NC = 2
NS = 16
NT = NC * NS
LANES = 16

ROWS_T = 320
TRASH = ROWS_T
NP = NT * ROWS_T
W = 2000
EC = 64
LAST_ROWS = N - (NT - 1) * ROWS_T


def _tc_body(obs_ref, wfc_ref, bfc_ref, w12_ref, ba_ref, feat_ref, pq_ref):
    f = lax.dot_general(
        obs_ref[...], wfc_ref[...], (((1,), (1,)), ((), ())),
        preferred_element_type=jnp.float32, precision=lax.Precision.HIGHEST)
    f = f + bfc_ref[...]
    feat_ref[...] = f
    pq = lax.dot_general(
        w12_ref[...], f, (((1,), (1,)), ((), ())),
        preferred_element_type=jnp.float32, precision=lax.Precision.HIGHEST)
    b = ba_ref[0, 0]
    rowmask = lax.broadcasted_iota(jnp.int32, (8, N), 0) == 0
    pq = pq + jnp.where(rowmask, b, 0.0)
    pq_ref[...] = jnp.concatenate(
        [pq, jnp.zeros((8, NP - N), jnp.float32)], axis=1)


def _tc_features(obs, wfc, bfc, w12, ba):
    return pl.pallas_call(
        _tc_body,
        in_specs=[
            pl.BlockSpec((N, D), lambda: (0, 0)),
            pl.BlockSpec((D, D), lambda: (0, 0)),
            pl.BlockSpec((1, D), lambda: (0, 0)),
            pl.BlockSpec((8, D), lambda: (0, 0)),
            pl.BlockSpec(memory_space=pltpu.SMEM),
        ],
        out_specs=[
            pl.BlockSpec((N, D), lambda: (0, 0)),
            pl.BlockSpec((8, NP), lambda: (0, 0)),
        ],
        out_shape=[
            jax.ShapeDtypeStruct((N, D), jnp.float32),
            jax.ShapeDtypeStruct((8, NP), jnp.float32),
        ],
    )(obs, wfc, bfc, w12, ba)


@functools.partial(
    pl.kernel,
    out_type=jax.ShapeDtypeStruct((N, D), jnp.float32),
    mesh=plsc.VectorSubcoreMesh(core_axis_name="c", subcore_axis_name="s"),
    compiler_params=pltpu.CompilerParams(needs_layout_passes=False),
    scratch_types=[
        pltpu.VMEM((ROWS_T + 1, D), jnp.float32),
        pltpu.VMEM((NP,), jnp.float32),
        pltpu.VMEM((NP,), jnp.float32),
        pltpu.VMEM((W,), jnp.int32),
        pltpu.VMEM((W,), jnp.int32),
        pltpu.VMEM((W + LANES,), jnp.int32),
        pltpu.VMEM((EC, D), jnp.float32),
        pltpu.VMEM((EC,), jnp.float32),
        pltpu.VMEM((EC,), jnp.int32),
        pltpu.VMEM((EC,), jnp.int32),
        pltpu.SemaphoreType.DMA,
    ],
)
def _sc_gat(feat_hbm, pq_hbm, src_hbm, dst_hbm, out_hbm,
            acc, p_v, q_v, src_w, dst_w, sel_v, rows_v, alpha_v, dloc_v,
            gidx_v, sem):
    c = lax.axis_index("c")
    s = lax.axis_index("s")
    wid = c * NS + s
    lo = wid * ROWS_T
    iota = lax.broadcasted_iota(jnp.int32, (LANES,), 0)
    zeros16 = jnp.zeros((LANES,), jnp.int32)

    def zrow(r, carry):
        def zcol(j, carry2):
            acc[r, pl.ds(j * LANES, LANES)] = jnp.zeros((LANES,), jnp.float32)
            return carry2
        return lax.fori_loop(0, D // LANES, zcol, carry)
    lax.fori_loop(0, ROWS_T + 1, zrow, 0)

    pltpu.sync_copy(pq_hbm.at[0], p_v)
    pltpu.sync_copy(pq_hbm.at[1], q_v)

    def window(w, carry):
        woff = w * W
        pltpu.sync_copy(src_hbm.at[pl.ds(woff, W)], src_w)
        pltpu.sync_copy(dst_hbm.at[pl.ds(woff, W)], dst_w)

        def scan(g, cnt):
            d16 = dst_w[pl.ds(g * LANES, LANES)]
            m = (d16 >= lo) & (d16 < lo + ROWS_T)
            plsc.store_compressed(
                sel_v.at[pl.ds(cnt, LANES)], g * LANES + iota, mask=m)
            return cnt + jnp.sum(m.astype(jnp.int32))
        cnt = lax.fori_loop(0, W // LANES, scan, jnp.int32(0))

        def chunk(j, carry2):
            cbase = j * EC
            for g in range(EC // LANES):
                lane0 = cbase + g * LANES
                valid = (lane0 + iota) < cnt
                pos16 = sel_v[pl.ds(lane0, LANES)]
                pos16 = jnp.where(valid, pos16, 0)
                s16 = plsc.load_gather(src_w, [pos16])
                d16 = plsc.load_gather(dst_w, [pos16])
                pg = plsc.load_gather(p_v, [s16])
                qg = plsc.load_gather(q_v, [d16])
                a = 1.0 / (1.0 + jnp.exp(-(pg + qg)))
                a = jnp.where(valid, a, 0.0)
                dl16 = jnp.where(valid, d16 - lo, TRASH)
                alpha_v[pl.ds(g * LANES, LANES)] = a
                dloc_v[pl.ds(g * LANES, LANES)] = dl16
                gidx_v[pl.ds(g * LANES, LANES)] = s16
            pltpu.async_copy(feat_hbm.at[gidx_v], rows_v, sem).wait()

            def srow(e, carry3):
                ab = plsc.load_gather(alpha_v, [zeros16 + e])
                dlb = plsc.load_gather(dloc_v, [zeros16 + e])
                for jj in range(D // LANES):
                    v = rows_v[e, pl.ds(jj * LANES, LANES)] * ab
                    plsc.addupdate_scatter(acc, [dlb, jj * LANES + iota], v)
                return carry3
            lax.fori_loop(0, EC, srow, 0)
            return carry2
        nch = (cnt + (EC - 1)) // EC
        lax.fori_loop(0, nch, chunk, 0)
        return carry
    lax.fori_loop(0, E // W, window, 0)

    @pl.when(wid < NT - 1)
    def _full():
        pltpu.sync_copy(acc.at[pl.ds(0, ROWS_T)],
                        out_hbm.at[pl.ds(lo, ROWS_T)])

    @pl.when(wid == NT - 1)
    def _last():
        pltpu.sync_copy(acc.at[pl.ds(0, LAST_ROWS)],
                        out_hbm.at[pl.ds(lo, LAST_ROWS)])


def kernel(observations, edge_index, W_fc, b_fc, W_attn, b_attn):
    w1 = W_attn[0, :D]
    w2 = W_attn[0, D:]
    w12 = jnp.zeros((8, D), jnp.float32).at[0].set(w1).at[1].set(w2)
    feat, pq = _tc_features(
        observations, W_fc, b_fc.reshape(1, D), w12, b_attn.reshape(1, 1))
    src = edge_index[0]
    dst = edge_index[1]
    return _sc_gat(feat, pq, src, dst)


# speedup vs baseline: 3.1323x; 2.2291x over previous
import functools

import jax
import jax.numpy as jnp
from jax import lax
from jax.experimental import pallas as pl
from jax.experimental.pallas import tpu as pltpu
from jax.experimental.pallas import tpu_sc as plsc

N = 10000
E = 160000
D = 256

NC = 2
NS = 16
NT = NC * NS
LANES = 16

ROWS_T = 320
TRASH = ROWS_T
NP = NT * ROWS_T
W = 1600
NW = E // W
EC = 32
CAP = W + 4 * LANES
LAST_ROWS = N - (NT - 1) * ROWS_T


def _tc_body(obs_ref, wfc_ref, bfc_ref, w12_ref, ba_ref, feat_ref, pq_ref):
    f = lax.dot_general(
        obs_ref[...], wfc_ref[...], (((1,), (1,)), ((), ())),
        preferred_element_type=jnp.float32, precision=lax.Precision.HIGHEST)
    f = f + bfc_ref[...]
    feat_ref[...] = f
    pq = lax.dot_general(
        w12_ref[...], f, (((1,), (1,)), ((), ())),
        preferred_element_type=jnp.float32, precision=lax.Precision.HIGHEST)
    b = ba_ref[0, 0]
    rowmask = lax.broadcasted_iota(jnp.int32, (8, N), 0) == 0
    pq = pq + jnp.where(rowmask, b, 0.0)
    pq_ref[...] = jnp.concatenate(
        [pq, jnp.zeros((8, NP - N), jnp.float32)], axis=1)


def _tc_features(obs, wfc, bfc, w12, ba):
    return pl.pallas_call(
        _tc_body,
        in_specs=[
            pl.BlockSpec((N, D), lambda: (0, 0)),
            pl.BlockSpec((D, D), lambda: (0, 0)),
            pl.BlockSpec((1, D), lambda: (0, 0)),
            pl.BlockSpec((8, D), lambda: (0, 0)),
            pl.BlockSpec(memory_space=pltpu.SMEM),
        ],
        out_specs=[
            pl.BlockSpec((N, D), lambda: (0, 0)),
            pl.BlockSpec((8, NP), lambda: (0, 0)),
        ],
        out_shape=[
            jax.ShapeDtypeStruct((N, D), jnp.float32),
            jax.ShapeDtypeStruct((8, NP), jnp.float32),
        ],
    )(obs, wfc, bfc, w12, ba)


@functools.partial(
    pl.kernel,
    out_type=jax.ShapeDtypeStruct((N, D), jnp.float32),
    mesh=plsc.VectorSubcoreMesh(core_axis_name="c", subcore_axis_name="s"),
    compiler_params=pltpu.CompilerParams(needs_layout_passes=False),
    scratch_types=[
        pltpu.VMEM((ROWS_T + 1, D), jnp.float32),
        pltpu.VMEM((NP,), jnp.float32),
        pltpu.VMEM((NP,), jnp.float32),
        pltpu.VMEM((2 * W,), jnp.int32),
        pltpu.VMEM((2 * W,), jnp.int32),
        pltpu.VMEM((CAP,), jnp.int32),
        pltpu.VMEM((CAP,), jnp.int32),
        pltpu.VMEM((2 * EC, D), jnp.float32),
        pltpu.VMEM((2 * EC,), jnp.float32),
        pltpu.VMEM((2 * EC,), jnp.int32),
        pltpu.VMEM((2 * EC,), jnp.int32),
        pltpu.SemaphoreType.DMA((2,)),
        pltpu.SemaphoreType.DMA((2,)),
    ],
)
def _sc_gat(feat_hbm, pq_hbm, src_hbm, dst_hbm, out_hbm,
            acc, p_v, q_v, src_w, dst_w, ssel, dsel, rows_v, alpha_v, dloc_v,
            gidx_v, wsem, gsem):
    c = lax.axis_index("c")
    s = lax.axis_index("s")
    wid = c * NS + s
    lo = wid * ROWS_T
    hi = lo + ROWS_T
    iota = lax.broadcasted_iota(jnp.int32, (LANES,), 0)
    zeros16 = jnp.zeros((LANES,), jnp.int32)
    cols = [jj * LANES + iota for jj in range(D // LANES)]

    def zrow(r, carry):
        def zcol(j, carry2):
            acc[r, pl.ds(j * LANES, LANES)] = jnp.zeros((LANES,), jnp.float32)
            return carry2
        return lax.fori_loop(0, D // LANES, zcol, carry)
    lax.fori_loop(0, ROWS_T + 1, zrow, 0)

    pltpu.sync_copy(pq_hbm.at[0], p_v)
    pltpu.sync_copy(pq_hbm.at[1], q_v)

    def start_gather(par):
        pltpu.async_copy(feat_hbm.at[gidx_v.at[pl.ds(par * EC, EC)]],
                         rows_v.at[pl.ds(par * EC, EC)], gsem.at[par])

    def accumulate(par):
        pltpu.make_async_copy(feat_hbm.at[gidx_v.at[pl.ds(par * EC, EC)]],
                              rows_v.at[pl.ds(par * EC, EC)],
                              gsem.at[par]).wait()

        def srow(e, carry):
            ab = plsc.load_gather(alpha_v, [zeros16 + (par * EC + e)])
            dlb = plsc.load_gather(dloc_v, [zeros16 + (par * EC + e)])
            for jj in range(D // LANES):
                v = rows_v[par * EC + e, pl.ds(jj * LANES, LANES)] * ab
                plsc.addupdate_scatter(acc, [dlb, cols[jj]], v)
            return carry
        lax.fori_loop(0, EC, srow, 0)

    def prep(par, base):
        for g in range(EC // LANES):
            s16 = ssel[pl.ds(base + g * LANES, LANES)]
            d16 = dsel[pl.ds(base + g * LANES, LANES)]
            pg = plsc.load_gather(p_v, [s16])
            qg = plsc.load_gather(q_v, [d16])
            a = 1.0 / (1.0 + jnp.exp(-(pg + qg)))
            alpha_v[pl.ds(par * EC + g * LANES, LANES)] = a
            dloc_v[pl.ds(par * EC + g * LANES, LANES)] = d16 - lo
            gidx_v[pl.ds(par * EC + g * LANES, LANES)] = s16

    def prep_masked(par, npend):
        for g in range(EC // LANES):
            valid = (g * LANES + iota) < npend
            s16 = ssel[pl.ds(g * LANES, LANES)]
            d16 = dsel[pl.ds(g * LANES, LANES)]
            s16 = jnp.where(valid, s16, 0)
            d16 = jnp.where(valid, d16, lo)
            pg = plsc.load_gather(p_v, [s16])
            qg = plsc.load_gather(q_v, [d16])
            a = 1.0 / (1.0 + jnp.exp(-(pg + qg)))
            alpha_v[pl.ds(par * EC + g * LANES, LANES)] = jnp.where(
                valid, a, 0.0)
            dloc_v[pl.ds(par * EC + g * LANES, LANES)] = jnp.where(
                valid, d16 - lo, TRASH)
            gidx_v[pl.ds(par * EC + g * LANES, LANES)] = s16

    pltpu.async_copy(src_hbm.at[pl.ds(0, W)], src_w.at[pl.ds(0, W)],
                     wsem.at[0])
    pltpu.async_copy(dst_hbm.at[pl.ds(0, W)], dst_w.at[pl.ds(0, W)],
                     wsem.at[0])

    def window(w, st):
        pend0, inflight0, par0 = st
        cur = w % 2
        woff = w * W
        pltpu.make_async_copy(src_hbm.at[pl.ds(woff, W)],
                              src_w.at[pl.ds(cur * W, W)],
                              wsem.at[cur]).wait()
        pltpu.make_async_copy(dst_hbm.at[pl.ds(woff, W)],
                              dst_w.at[pl.ds(cur * W, W)],
                              wsem.at[cur]).wait()

        @pl.when(w + 1 < NW)
        def _prefetch():
            noff = (w + 1) * W
            pltpu.async_copy(src_hbm.at[pl.ds(noff, W)],
                             src_w.at[pl.ds((1 - cur) * W, W)],
                             wsem.at[1 - cur])
            pltpu.async_copy(dst_hbm.at[pl.ds(noff, W)],
                             dst_w.at[pl.ds((1 - cur) * W, W)],
                             wsem.at[1 - cur])

        def scan(g, pendc):
            s16 = src_w[pl.ds(cur * W + g * LANES, LANES)]
            d16 = dst_w[pl.ds(cur * W + g * LANES, LANES)]
            m = (d16 >= lo) & (d16 < hi)
            plsc.store_compressed(ssel.at[pl.ds(pendc, LANES)], s16, mask=m)
            plsc.store_compressed(dsel.at[pl.ds(pendc, LANES)], d16, mask=m)
            return pendc + plsc.all_reduce_population_count(m)[0]
        pend = lax.fori_loop(0, W // LANES, scan, pend0)

        def pcond(pst):
            return pst[0] >= EC

        def pbody(pst):
            pendc, basec, inflightc, parc = pst
            prep(parc, basec)
            start_gather(parc)

            @pl.when(inflightc == 1)
            def _drain():
                accumulate(1 - parc)
            return (pendc - EC, basec + EC, jnp.int32(1), 1 - parc)
        pend, base, inflight, par = lax.while_loop(
            pcond, pbody, (pend, jnp.int32(0), inflight0, par0))

        for k in range(EC // LANES):
            idx16 = base + k * LANES + iota
            sv = plsc.load_gather(ssel, [idx16])
            dv = plsc.load_gather(dsel, [idx16])
            ssel[pl.ds(k * LANES, LANES)] = sv
            dsel[pl.ds(k * LANES, LANES)] = dv
        return (pend, inflight, par)

    pend, inflight, par = lax.fori_loop(
        0, NW, window, (jnp.int32(0), jnp.int32(0), jnp.int32(0)))

    @pl.when(pend > 0)
    def _tail():
        prep_masked(par, pend)
        start_gather(par)

        @pl.when(inflight == 1)
        def _drain_prev():
            accumulate(1 - par)
        accumulate(par)

    @pl.when((pend == 0) & (inflight == 1))
    def _drain_last():
        accumulate(1 - par)

    @pl.when(wid < NT - 1)
    def _full():
        pltpu.sync_copy(acc.at[pl.ds(0, ROWS_T)],
                        out_hbm.at[pl.ds(lo, ROWS_T)])

    @pl.when(wid == NT - 1)
    def _last():
        pltpu.sync_copy(acc.at[pl.ds(0, LAST_ROWS)],
                        out_hbm.at[pl.ds(lo, LAST_ROWS)])


def kernel(observations, edge_index, W_fc, b_fc, W_attn, b_attn):
    w1 = W_attn[0, :D]
    w2 = W_attn[0, D:]
    w12 = jnp.zeros((8, D), jnp.float32).at[0].set(w1).at[1].set(w2)
    feat, pq = _tc_features(
        observations, W_fc, b_fc.reshape(1, D), w12, b_attn.reshape(1, 1))
    src = edge_index[0]
    dst = edge_index[1]
    return _sc_gat(feat, pq, src, dst)


# speedup vs baseline: 3.2139x; 1.0261x over previous
import functools

import jax
import jax.numpy as jnp
from jax import lax
from jax.experimental import pallas as pl
from jax.experimental.pallas import tpu as pltpu
from jax.experimental.pallas import tpu_sc as plsc

N = 10000
E = 160000
D = 256

NC = 2
NS = 16
NT = NC * NS
LANES = 16

ROWS_T = 320
TRASH = ROWS_T
NP = NT * ROWS_T
W = 1600
NW = E // W
EC = 32
CAP = W + 4 * LANES
LAST_ROWS = N - (NT - 1) * ROWS_T


PSH = 16384


def _tc_body(obs_ref, wfc_ref, bfc_ref, w12_ref, ba_ref, ei_ref,
             feat_ref, pq_ref, pk_ref):
    f = lax.dot_general(
        obs_ref[...], wfc_ref[...], (((1,), (1,)), ((), ())),
        preferred_element_type=jnp.float32, precision=lax.Precision.HIGHEST)
    f = f + bfc_ref[...]
    feat_ref[...] = f
    pq = lax.dot_general(
        w12_ref[...], f, (((1,), (1,)), ((), ())),
        preferred_element_type=jnp.float32, precision=lax.Precision.HIGHEST)
    b = ba_ref[0, 0]
    rowmask = lax.broadcasted_iota(jnp.int32, (8, N), 0) == 0
    pq = pq + jnp.where(rowmask, b, 0.0)
    pq_ref[...] = jnp.concatenate(
        [pq, jnp.zeros((8, NP - N), jnp.float32)], axis=1)
    ei = ei_ref[...]
    pk_ref[...] = ei[0:1, :] * PSH + ei[1:2, :]


def _tc_features(obs, wfc, bfc, w12, ba, ei):
    return pl.pallas_call(
        _tc_body,
        in_specs=[
            pl.BlockSpec((N, D), lambda: (0, 0)),
            pl.BlockSpec((D, D), lambda: (0, 0)),
            pl.BlockSpec((1, D), lambda: (0, 0)),
            pl.BlockSpec((8, D), lambda: (0, 0)),
            pl.BlockSpec(memory_space=pltpu.SMEM),
            pl.BlockSpec((2, E), lambda: (0, 0)),
        ],
        out_specs=[
            pl.BlockSpec((N, D), lambda: (0, 0)),
            pl.BlockSpec((8, NP), lambda: (0, 0)),
            pl.BlockSpec((1, E), lambda: (0, 0)),
        ],
        out_shape=[
            jax.ShapeDtypeStruct((N, D), jnp.float32),
            jax.ShapeDtypeStruct((8, NP), jnp.float32),
            jax.ShapeDtypeStruct((1, E), jnp.int32),
        ],
    )(obs, wfc, bfc, w12, ba, ei)


SCAN_UNROLL = 4


@functools.partial(
    pl.kernel,
    out_type=jax.ShapeDtypeStruct((N, D), jnp.float32),
    mesh=plsc.VectorSubcoreMesh(core_axis_name="c", subcore_axis_name="s"),
    compiler_params=pltpu.CompilerParams(needs_layout_passes=False),
    scratch_types=[
        pltpu.VMEM((ROWS_T + 1, D), jnp.float32),
        pltpu.VMEM((NP,), jnp.float32),
        pltpu.VMEM((NP,), jnp.float32),
        pltpu.VMEM((2 * W,), jnp.int32),
        pltpu.VMEM((CAP,), jnp.int32),
        pltpu.VMEM((2 * EC, D), jnp.float32),
        pltpu.VMEM((2 * EC,), jnp.float32),
        pltpu.VMEM((2 * EC,), jnp.int32),
        pltpu.VMEM((2 * EC,), jnp.int32),
        pltpu.SemaphoreType.DMA((2,)),
        pltpu.SemaphoreType.DMA((2,)),
    ],
)
def _sc_gat(feat_hbm, pq_hbm, pk_hbm, out_hbm,
            acc, p_v, q_v, pk_w, esel, rows_v, alpha_v, dloc_v,
            gidx_v, wsem, gsem):
    c = lax.axis_index("c")
    s = lax.axis_index("s")
    wid = c * NS + s
    lo = wid * ROWS_T
    hi = lo + ROWS_T
    iota = lax.broadcasted_iota(jnp.int32, (LANES,), 0)
    zerosf = jnp.zeros((LANES,), jnp.float32)
    zeros16 = jnp.zeros((LANES,), jnp.int32)
    cols = [jj * LANES + iota for jj in range(D // LANES)]

    def zrow(r, carry):
        for j in range(D // LANES):
            acc[r, pl.ds(j * LANES, LANES)] = zerosf
        return carry
    lax.fori_loop(0, ROWS_T + 1, zrow, 0)

    pltpu.sync_copy(pq_hbm.at[0], p_v)
    pltpu.sync_copy(pq_hbm.at[1], q_v)

    def start_gather(par):
        pltpu.async_copy(feat_hbm.at[gidx_v.at[pl.ds(par * EC, EC)]],
                         rows_v.at[pl.ds(par * EC, EC)], gsem.at[par])

    def accumulate(par):
        pltpu.make_async_copy(feat_hbm.at[gidx_v.at[pl.ds(par * EC, EC)]],
                              rows_v.at[pl.ds(par * EC, EC)],
                              gsem.at[par]).wait()

        def srow(e2, carry):
            for u in range(2):
                pe = par * EC + e2 * 2 + u
                ab = plsc.load_gather(alpha_v, [zeros16 + pe])
                dlb = plsc.load_gather(dloc_v, [zeros16 + pe])
                for jj in range(D // LANES):
                    v = rows_v[pe, pl.ds(jj * LANES, LANES)] * ab
                    plsc.addupdate_scatter(acc, [dlb, cols[jj]], v)
            return carry
        lax.fori_loop(0, EC // 2, srow, 0)

    def prep_groups(par, base, npend):
        for g in range(EC // LANES):
            pk16 = esel[pl.ds(base + g * LANES, LANES)]
            s16 = lax.shift_right_logical(pk16, 14)
            d16 = pk16 & (PSH - 1)
            if npend is not None:
                valid = (g * LANES + iota) < npend
                s16 = jnp.where(valid, s16, 0)
                d16 = jnp.where(valid, d16, lo)
            pg = plsc.load_gather(p_v, [s16])
            qg = plsc.load_gather(q_v, [d16])
            a = 1.0 / (1.0 + jnp.exp(-(pg + qg)))
            dl16 = d16 - lo
            if npend is not None:
                a = jnp.where(valid, a, 0.0)
                dl16 = jnp.where(valid, dl16, TRASH)
            alpha_v[pl.ds(par * EC + g * LANES, LANES)] = a
            dloc_v[pl.ds(par * EC + g * LANES, LANES)] = dl16
            gidx_v[pl.ds(par * EC + g * LANES, LANES)] = s16

    def prep(par, base):
        prep_groups(par, base, None)

    def prep_masked(par, npend):
        prep_groups(par, jnp.int32(0), npend)

    pltpu.async_copy(pk_hbm.at[pl.ds(0, W)], pk_w.at[pl.ds(0, W)],
                     wsem.at[0])

    def window(w, st):
        pend0, inflight0, par0 = st
        cur = w % 2
        woff = w * W
        pltpu.make_async_copy(pk_hbm.at[pl.ds(woff, W)],
                              pk_w.at[pl.ds(cur * W, W)],
                              wsem.at[cur]).wait()

        @pl.when(w + 1 < NW)
        def _prefetch():
            noff = (w + 1) * W
            pltpu.async_copy(pk_hbm.at[pl.ds(noff, W)],
                             pk_w.at[pl.ds((1 - cur) * W, W)],
                             wsem.at[1 - cur])

        def scan(g, pendc):
            for u in range(SCAN_UNROLL):
                pk16 = pk_w[pl.ds(cur * W + (g * SCAN_UNROLL + u) * LANES,
                                  LANES)]
                d16 = pk16 & (PSH - 1)
                m = (d16 >= lo) & (d16 < hi)
                plsc.store_compressed(esel.at[pl.ds(pendc, LANES)], pk16,
                                      mask=m)
                pendc = pendc + plsc.all_reduce_population_count(m)[0]
            return pendc
        pend = lax.fori_loop(0, W // (LANES * SCAN_UNROLL), scan, pend0)

        def pcond(pst):
            return pst[0] >= EC

        def pbody(pst):
            pendc, basec, inflightc, parc = pst
            prep(parc, basec)
            start_gather(parc)

            @pl.when(inflightc == 1)
            def _drain():
                accumulate(1 - parc)
            return (pendc - EC, basec + EC, jnp.int32(1), 1 - parc)
        pend, base, inflight, par = lax.while_loop(
            pcond, pbody, (pend, jnp.int32(0), inflight0, par0))

        for k in range(EC // LANES):
            idx16 = base + k * LANES + iota
            ev = plsc.load_gather(esel, [idx16])
            esel[pl.ds(k * LANES, LANES)] = ev
        return (pend, inflight, par)

    pend, inflight, par = lax.fori_loop(
        0, NW, window, (jnp.int32(0), jnp.int32(0), jnp.int32(0)))

    @pl.when(pend > 0)
    def _tail():
        prep_masked(par, pend)
        start_gather(par)

        @pl.when(inflight == 1)
        def _drain_prev():
            accumulate(1 - par)
        accumulate(par)

    @pl.when((pend == 0) & (inflight == 1))
    def _drain_last():
        accumulate(1 - par)

    @pl.when(wid < NT - 1)
    def _full():
        pltpu.sync_copy(acc.at[pl.ds(0, ROWS_T)],
                        out_hbm.at[pl.ds(lo, ROWS_T)])

    @pl.when(wid == NT - 1)
    def _last():
        pltpu.sync_copy(acc.at[pl.ds(0, LAST_ROWS)],
                        out_hbm.at[pl.ds(lo, LAST_ROWS)])


def kernel(observations, edge_index, W_fc, b_fc, W_attn, b_attn):
    w1 = W_attn[0, :D]
    w2 = W_attn[0, D:]
    w12 = jnp.zeros((8, D), jnp.float32).at[0].set(w1).at[1].set(w2)
    feat, pq, pk = _tc_features(
        observations, W_fc, b_fc.reshape(1, D), w12, b_attn.reshape(1, 1),
        edge_index)
    return _sc_gat(feat, pq, pk.reshape(E))
